# Initial kernel scaffold; baseline (speedup 1.0000x reference)
import jax
import jax.numpy as jnp
from jax.experimental import pallas as pl


def kernel(x_order, x_rider, edge_index, edge_attr, W_src, W_dst, att_src, att_dst, bias_gat, W_op, b_op, W_g1, b_g1, W_g2, b_g2):
    raise NotImplementedError("write your pallas kernel here")


# speedup vs baseline: 3.7023x; 3.7023x over previous
import functools

import jax
import jax.numpy as jnp
from jax import lax
from jax.experimental import pallas as pl
from jax.experimental.pallas import tpu as pltpu
from jax.experimental.pallas import tpu_sc as plsc

N_ORD = 10000
N_RID = 10000
EDGES = 160000
D_ORD = 256
D_RID = 128
HID = 256
HH = 128
ACCW = 144
DENC = 128
NRPAD = 10240
RPS = NRPAD // 16
EPAD = 163840
EPS3 = EPAD // 32
CH1 = 80
NCH1 = (EDGES // 16) // CH1
CH3 = 64
NCH3 = EPS3 // CH3
NSTRIP = 80
CH0 = 128
NCH0 = EPS3 // CH0


def _dense_body(xo_ref, xr_ref, ws_ref, wo_ref, bo_ref, avs_ref, wd_ref,
                avd_ref, hlo_ref, hhi_ref, op_ref, asrc_ref, adst_ref):
    xo = xo_ref[...]
    h = jnp.dot(xo, ws_ref[...], preferred_element_type=jnp.float32)
    hlo_ref[...] = h[:, :HH]
    hhi_ref[...] = h[:, HH:]
    asrc_ref[...] = jnp.sum(h * avs_ref[...], axis=1)[None, None, :]
    op_ref[...] = jnp.dot(xo, wo_ref[...], preferred_element_type=jnp.float32) + bo_ref[...]
    wd = jnp.sum(wd_ref[...] * avd_ref[...], axis=1)
    adst_ref[...] = jnp.sum(xr_ref[...] * wd[None, :], axis=1)[None, None, :]


def _gate_body(ea_ref, w1_ref, b1_ref, w2_ref, b2_ref, g_ref):
    hg = jnp.maximum(
        jnp.dot(ea_ref[...], w1_ref[...], preferred_element_type=jnp.float32)
        + b1_ref[...], 0.0)
    z = jnp.dot(hg, w2_ref[...], preferred_element_type=jnp.float32) + b2_ref[...]
    g_ref[...] = jax.nn.sigmoid(z)


@functools.partial(
    pl.kernel,
    out_type=jax.ShapeDtypeStruct((EPAD,), jnp.float32),
    mesh=plsc.VectorSubcoreMesh(core_axis_name="c", subcore_axis_name="s"),
    compiler_params=pltpu.CompilerParams(use_tc_tiling_on_sc=False, needs_layout_passes=False),
    scratch_types=[
        pltpu.VMEM((N_ORD,), jnp.float32),
        pltpu.VMEM((N_RID,), jnp.float32),
        pltpu.VMEM((CH0,), jnp.int32),
        pltpu.VMEM((CH0,), jnp.int32),
        pltpu.VMEM((CH0,), jnp.float32),
    ],
)
def _phase0(oi_hbm, ri_hbm, asrc_hbm, adst_hbm, ex_hbm,
            asrc_v, adst_v, oi_c, ri_c, ex_c):
    c = lax.axis_index("c")
    s = lax.axis_index("s")
    wid = s * 2 + c
    pltpu.sync_copy(asrc_hbm, asrc_v)
    pltpu.sync_copy(adst_hbm, adst_v)

    def chunk(g, carry):
        base = wid * EPS3 + g * CH0
        pltpu.sync_copy(oi_hbm.at[pl.ds(base, CH0)], oi_c)
        pltpu.sync_copy(ri_hbm.at[pl.ds(base, CH0)], ri_c)
        for k in range(CH0 // 16):
            oiv = oi_c[pl.ds(k * 16, 16)]
            riv = ri_c[pl.ds(k * 16, 16)]
            av = plsc.load_gather(asrc_v, [oiv]) + plsc.load_gather(adst_v, [riv])
            al = jnp.maximum(av, 0.2 * av)
            ex_c[pl.ds(k * 16, 16)] = jnp.exp(al)
        pltpu.sync_copy(ex_c, ex_hbm.at[pl.ds(base, CH0)])
        return carry

    lax.fori_loop(0, NCH0, chunk, 0)


@functools.partial(
    pl.kernel,
    out_type=[jax.ShapeDtypeStruct((NRPAD, ACCW), jnp.float32),
              jax.ShapeDtypeStruct((NRPAD, ACCW), jnp.float32)],
    mesh=plsc.VectorSubcoreMesh(core_axis_name="c", subcore_axis_name="s"),
    compiler_params=pltpu.CompilerParams(use_tc_tiling_on_sc=False, needs_layout_passes=False),
    scratch_types=[
        pltpu.VMEM((CH1,), jnp.int32),
        pltpu.VMEM((CH1,), jnp.int32),
        pltpu.VMEM((CH1,), jnp.float32),
        pltpu.VMEM((CH1, HH), jnp.float32),
        pltpu.VMEM((CH1, ACCW), jnp.float32),
        pltpu.VMEM((NSTRIP, ACCW), jnp.float32),
        pltpu.VMEM((HH,), jnp.float32),
        pltpu.VMEM_SHARED((NRPAD, ACCW), jnp.float32),
        pltpu.SemaphoreType.DMA,
    ],
)
def _phase1(oi_hbm, ri_hbm, ex_hbm, hlo_hbm, hhi_hbm, bias_hbm,
            relo_hbm, rehi_hbm,
            oi_c, ri_c, ex_c, rows, staged, nbuf, bias_v,
            acc, sem):
    c = lax.axis_index("c")
    s = lax.axis_index("s")

    @pl.when(c == 0)
    def _():
        pltpu.sync_copy(bias_hbm.at[pl.ds(0, HH)], bias_v)

    @pl.when(c == 1)
    def _():
        pltpu.sync_copy(bias_hbm.at[pl.ds(HH, HH)], bias_v)

    zv = jnp.zeros((16,), jnp.float32)

    def zrow(r, carry):
        for k in range(ACCW // 16):
            nbuf[r, pl.ds(k * 16, 16)] = zv
        return carry

    lax.fori_loop(0, NSTRIP, zrow, 0)
    for t in range(RPS // NSTRIP):
        pltpu.sync_copy(nbuf, acc.at[pl.ds(s * RPS + t * NSTRIP, NSTRIP)])
    plsc.subcore_barrier()

    lane0 = lax.iota(jnp.int32, 16) == 0

    def chunk(g, carry):
        base = s * (EDGES // 16) + g * CH1
        pltpu.sync_copy(oi_hbm.at[pl.ds(base, CH1)], oi_c)
        pltpu.sync_copy(ri_hbm.at[pl.ds(base, CH1)], ri_c)
        pltpu.sync_copy(ex_hbm.at[pl.ds(base, CH1)], ex_c)

        @pl.when(c == 0)
        def _():
            pltpu.async_copy(hlo_hbm.at[oi_c], rows, sem).wait()

        @pl.when(c == 1)
        def _():
            pltpu.async_copy(hhi_hbm.at[oi_c], rows, sem).wait()

        for k in range(CH1 // 16):
            exv = ex_c[pl.ds(k * 16, 16)]
            for e16 in range(16):
                e = k * 16 + e16
                exs = exv[e16]
                for q in range(HH // 16):
                    staged[e, pl.ds(q * 16, 16)] = rows[e, pl.ds(q * 16, 16)] * exs
                staged[e, pl.ds(DENC, 16)] = jnp.where(lane0, exs, 0.0)

        pltpu.sync_copy(staged, acc.at[ri_c], add=True)
        return carry

    lax.fori_loop(0, NCH1, chunk, 0)
    plsc.subcore_barrier()

    for t in range(RPS // NSTRIP):
        r0 = s * RPS + t * NSTRIP
        pltpu.sync_copy(acc.at[pl.ds(r0, NSTRIP)], nbuf)

        def nrow(r, carry):
            denv = nbuf[r, pl.ds(DENC, 16)]
            den = denv[0] + 1e-16
            inv = 1.0 / (jnp.zeros((16,), jnp.float32) + den)
            for k in range(HH // 16):
                nbuf[r, pl.ds(k * 16, 16)] = (
                    nbuf[r, pl.ds(k * 16, 16)] * inv + bias_v[pl.ds(k * 16, 16)])
            return carry

        lax.fori_loop(0, NSTRIP, nrow, 0)

        @pl.when(c == 0)
        def _():
            pltpu.sync_copy(nbuf, relo_hbm.at[pl.ds(r0, NSTRIP)])

        @pl.when(c == 1)
        def _():
            pltpu.sync_copy(nbuf, rehi_hbm.at[pl.ds(r0, NSTRIP)])


@functools.partial(
    pl.kernel,
    out_type=jax.ShapeDtypeStruct((EPAD,), jnp.float32),
    mesh=plsc.VectorSubcoreMesh(core_axis_name="c", subcore_axis_name="s"),
    compiler_params=pltpu.CompilerParams(use_tc_tiling_on_sc=False, needs_layout_passes=False),
    scratch_types=[
        pltpu.VMEM((CH3,), jnp.int32),
        pltpu.VMEM((CH3,), jnp.int32),
        pltpu.VMEM((CH3,), jnp.float32),
        pltpu.VMEM((CH3, HID), jnp.float32),
        pltpu.VMEM((CH3, ACCW), jnp.float32),
        pltpu.VMEM((CH3, ACCW), jnp.float32),
        pltpu.VMEM((CH3,), jnp.float32),
        pltpu.SemaphoreType.DMA,
    ],
)
def _phase3(oi_hbm, ri_hbm, gate_hbm, op_hbm, relo_hbm, rehi_hbm, raw_hbm,
            oi_c, ri_c, g_c, opr, rlo, rhi, ob, sem):
    c = lax.axis_index("c")
    s = lax.axis_index("s")
    wid = s * 2 + c

    def chunk(g, carry):
        base = wid * EPS3 + g * CH3
        pltpu.sync_copy(oi_hbm.at[pl.ds(base, CH3)], oi_c)
        pltpu.sync_copy(ri_hbm.at[pl.ds(base, CH3)], ri_c)
        pltpu.sync_copy(gate_hbm.at[pl.ds(base, CH3)], g_c)
        cp1 = pltpu.async_copy(op_hbm.at[oi_c], opr, sem)
        cp2 = pltpu.async_copy(relo_hbm.at[ri_c], rlo, sem)
        cp3 = pltpu.async_copy(rehi_hbm.at[ri_c], rhi, sem)
        cp1.wait()
        cp2.wait()
        cp3.wait()

        lane = lax.iota(jnp.int32, 16)
        for j in range(CH3 // 16):
            gv = g_c[pl.ds(j * 16, 16)]
            dots = jnp.zeros((16,), jnp.float32)
            for e16 in range(16):
                e = j * 16 + e16
                acc = opr[e, pl.ds(0, 16)] * rlo[e, pl.ds(0, 16)]
                for k in range(1, HH // 16):
                    acc = acc + opr[e, pl.ds(k * 16, 16)] * rlo[e, pl.ds(k * 16, 16)]
                for k in range(HH // 16):
                    acc = acc + opr[e, pl.ds(HH + k * 16, 16)] * rhi[e, pl.ds(k * 16, 16)]
                dots = jnp.where(lane == e16, jnp.sum(acc), dots)
            ob[pl.ds(j * 16, 16)] = dots * gv
        pltpu.sync_copy(ob, raw_hbm.at[pl.ds(base, CH3)])
        return carry

    lax.fori_loop(0, NCH3, chunk, 0)


def kernel(x_order, x_rider, edge_index, edge_attr, W_src, W_dst, att_src,
           att_dst, bias_gat, W_op, b_op, W_g1, b_g1, W_g2, b_g2):
    oi = edge_index[0]
    ri = edge_index[1]

    nb = N_ORD // 400
    hlo, hhi, op, asrc2, adst2 = pl.pallas_call(
        _dense_body,
        grid=(nb,),
        in_specs=[
            pl.BlockSpec((400, D_ORD), lambda i: (i, 0)),
            pl.BlockSpec((400, D_RID), lambda i: (i, 0)),
            pl.BlockSpec((D_ORD, HID), lambda i: (0, 0)),
            pl.BlockSpec((D_ORD, HID), lambda i: (0, 0)),
            pl.BlockSpec((1, HID), lambda i: (0, 0)),
            pl.BlockSpec((1, HID), lambda i: (0, 0)),
            pl.BlockSpec((D_RID, HID), lambda i: (0, 0)),
            pl.BlockSpec((1, HID), lambda i: (0, 0)),
        ],
        out_specs=[
            pl.BlockSpec((400, HH), lambda i: (i, 0)),
            pl.BlockSpec((400, HH), lambda i: (i, 0)),
            pl.BlockSpec((400, HID), lambda i: (i, 0)),
            pl.BlockSpec((1, 1, 400), lambda i: (i, 0, 0)),
            pl.BlockSpec((1, 1, 400), lambda i: (i, 0, 0)),
        ],
        out_shape=[
            jax.ShapeDtypeStruct((N_ORD, HH), jnp.float32),
            jax.ShapeDtypeStruct((N_ORD, HH), jnp.float32),
            jax.ShapeDtypeStruct((N_ORD, HID), jnp.float32),
            jax.ShapeDtypeStruct((nb, 1, 400), jnp.float32),
            jax.ShapeDtypeStruct((nb, 1, 400), jnp.float32),
        ],
    )(x_order, x_rider, W_src, W_op, b_op.reshape(1, -1),
      att_src.reshape(1, -1), W_dst, att_dst.reshape(1, -1))
    a_src = asrc2.reshape(-1)
    a_dst = adst2.reshape(-1)

    ea_p = jnp.pad(edge_attr, ((0, EPAD - EDGES), (0, 0)))
    gate2 = pl.pallas_call(
        _gate_body,
        grid=(EPAD // 2048,),
        in_specs=[
            pl.BlockSpec((2048, 16), lambda i: (i, 0)),
            pl.BlockSpec((16, 16), lambda i: (0, 0)),
            pl.BlockSpec((1, 16), lambda i: (0, 0)),
            pl.BlockSpec((16, 1), lambda i: (0, 0)),
            pl.BlockSpec((1, 1), lambda i: (0, 0)),
        ],
        out_specs=pl.BlockSpec((2048, 1), lambda i: (i, 0)),
        out_shape=jax.ShapeDtypeStruct((EPAD, 1), jnp.float32),
    )(ea_p, W_g1, b_g1.reshape(1, -1), W_g2, b_g2.reshape(1, -1))
    gate = gate2.reshape(-1)

    oi_p = jnp.pad(oi, (0, EPAD - EDGES))
    ri_p = jnp.pad(ri, (0, EPAD - EDGES))
    ex = _phase0(oi_p, ri_p, a_src, a_dst)
    relo, rehi = _phase1(oi, ri, ex, hlo, hhi, bias_gat)
    raw = _phase3(oi_p, ri_p, gate, op, relo, rehi)
    return raw[:EDGES]
